# baseline (speedup 1.0000x reference)
import functools

import jax
import jax.numpy as jnp
from jax import lax
from jax.experimental import pallas as pl
from jax.experimental.pallas import tpu as pltpu
from jax.experimental.pallas import tpu_sc as plsc

N = 10000
NP = 10240
E = 320000
EP = 327680
RG = 160
RD = 80
NB = NP // 256
ROWS_PER_TILE = NP // 16

F32 = jnp.float32
BF16 = jnp.bfloat16

_SC_MESH = dict(core_axis_name="c", subcore_axis_name="s")


@functools.cache
def _make_g_kernel(nc):
    W = 128
    split_edges = nc == 1
    blocks = [(0, 56), (56, 24)] if split_edges else [(0, 56), (56, 56), (112, 48)]
    RBMAX = 56
    n_out = 2 if split_edges else nc
    chunk_iters = 1 if split_edges else nc // 2

    @functools.partial(
        pl.kernel,
        out_type=jax.ShapeDtypeStruct((n_out, NP, W), F32),
        mesh=plsc.VectorSubcoreMesh(**_SC_MESH),
        cost_estimate=pl.CostEstimate(
            flops=0, transcendentals=0,
            bytes_accessed=nc * EP * W * 4 * 2),
        scratch_types=[
            pltpu.VMEM((RBMAX, 128), jnp.int32),
            pltpu.VMEM((RBMAX, 128), jnp.int32),
            pltpu.VMEM((128, W), F32),
            pltpu.VMEM((128, W), F32),
            pltpu.VMEM_SHARED((NP, W), F32),
            pltpu.SemaphoreType.DMA,
            pltpu.SemaphoreType.DMA,
            pltpu.SemaphoreType.DMA,
            pltpu.SemaphoreType.DMA,
        ],
    )
    def g_kernel(tab, srcr, dstr, zeros, out, src_v, dst_v, rows0, rows1,
                 accum, sem0, sem1, sem2, sem3):
        cid = lax.axis_index("c")
        sid = lax.axis_index("s")
        r0 = sid * ROWS_PER_TILE
        my_src = srcr.at[sid * 2 + cid] if split_edges else srcr.at[sid]
        my_dst = dstr.at[sid * 2 + cid] if split_edges else dstr.at[sid]
        for ci in range(chunk_iters):
            c = 0 if split_edges else 2 * ci + cid
            o = cid if split_edges else c
            tab_c = tab.at[c]
            pltpu.sync_copy(zeros.at[pl.ds(r0, ROWS_PER_TILE)],
                            accum.at[pl.ds(r0, ROWS_PER_TILE)])
            plsc.subcore_barrier()

            for off, rb in blocks:
                pltpu.sync_copy(my_src.at[pl.ds(off, rb)],
                                src_v.at[pl.ds(0, rb)])
                pltpu.sync_copy(my_dst.at[pl.ds(off, rb)],
                                dst_v.at[pl.ds(0, rb)])
                pltpu.async_copy(tab_c.at[src_v.at[0]], rows0, sem0)

                def body(i, _, rb=rb):
                    u = 2 * i

                    @pl.when(u > 0)
                    def _():
                        pltpu.make_async_copy(rows1, accum.at[dst_v.at[u - 1]],
                                              sem3).wait()

                    pltpu.async_copy(tab_c.at[src_v.at[u + 1]], rows1, sem1)
                    pltpu.make_async_copy(tab_c.at[src_v.at[u]], rows0,
                                          sem0).wait()
                    pltpu.async_copy(rows0, accum.at[dst_v.at[u]], sem2,
                                     add=True)
                    pltpu.make_async_copy(tab_c.at[src_v.at[u + 1]], rows1,
                                          sem1).wait()
                    pltpu.make_async_copy(rows0, accum.at[dst_v.at[u]],
                                          sem2).wait()

                    @pl.when(u + 2 < rb)
                    def _():
                        pltpu.async_copy(tab_c.at[src_v.at[u + 2]], rows0, sem0)

                    pltpu.async_copy(rows1, accum.at[dst_v.at[u + 1]], sem3,
                                     add=True)
                    return 0

                lax.fori_loop(0, rb // 2, body, 0)
                pltpu.make_async_copy(rows1, accum.at[dst_v.at[rb - 1]],
                                      sem3).wait()
            plsc.subcore_barrier()
            pltpu.sync_copy(accum.at[pl.ds(r0, ROWS_PER_TILE)],
                            out.at[o].at[pl.ds(r0, ROWS_PER_TILE)])
            plsc.subcore_barrier()

    return g_kernel


@functools.cache
def _make_deg_kernel():
    @functools.partial(
        pl.kernel,
        out_type=(jax.ShapeDtypeStruct((2, NP), F32),
                  jax.ShapeDtypeStruct((2, NP), F32)),
        mesh=plsc.VectorSubcoreMesh(**_SC_MESH),
        scratch_types=[
            pltpu.VMEM((RD, 128), jnp.int32),
            pltpu.VMEM((RD, 128), jnp.int32),
            pltpu.VMEM((128,), F32),
            pltpu.VMEM((128,), F32),
            pltpu.VMEM_SHARED((NP,), F32),
            pltpu.VMEM_SHARED((NP,), F32),
        ],
    )
    def deg_kernel(srcr, dstr, zeros1, deg_out, cs_out, src_v, dst_v,
                   mval, cval, acc_deg, acc_cs):
        cid = lax.axis_index("c")
        sid = lax.axis_index("s")
        wid = sid * 2 + cid
        r0 = sid * ROWS_PER_TILE
        pltpu.sync_copy(srcr.at[wid], src_v)
        pltpu.sync_copy(dstr.at[wid], dst_v)
        pltpu.sync_copy(zeros1.at[pl.ds(r0, ROWS_PER_TILE)],
                        acc_deg.at[pl.ds(r0, ROWS_PER_TILE)])
        pltpu.sync_copy(zeros1.at[pl.ds(r0, ROWS_PER_TILE)],
                        acc_cs.at[pl.ds(r0, ROWS_PER_TILE)])
        plsc.subcore_barrier()

        def body(j, _):
            for i in range(8):
                s = src_v[j, pl.ds(i * 16, 16)]
                d = dst_v[j, pl.ds(i * 16, 16)]
                m = jnp.where(s != d, F32(1.0), F32(0.0))
                mval[pl.ds(i * 16, 16)] = m
                cval[pl.ds(i * 16, 16)] = F32(1.0) - m
            pltpu.sync_copy(mval, acc_deg.at[src_v.at[j]], add=True)
            pltpu.sync_copy(cval, acc_cs.at[src_v.at[j]], add=True)
            return 0

        lax.fori_loop(0, RD, body, 0)
        plsc.subcore_barrier()
        pltpu.sync_copy(acc_deg.at[pl.ds(r0, ROWS_PER_TILE)],
                        deg_out.at[cid].at[pl.ds(r0, ROWS_PER_TILE)])
        pltpu.sync_copy(acc_cs.at[pl.ds(r0, ROWS_PER_TILE)],
                        cs_out.at[cid].at[pl.ds(r0, ROWS_PER_TILE)])

    return deg_kernel


def _prologue_body(deg2_ref, cs2_ref, x_ref, dinv_ref, cs_ref, xc_ref, xh_ref):
    deg = jnp.sum(deg2_ref[...], axis=0)
    cs = jnp.sum(cs2_ref[...], axis=0)
    dinv = jnp.where(deg > 0, lax.rsqrt(jnp.maximum(deg, F32(1.0))), F32(0.0))
    dinv_ref[...] = dinv
    cs_ref[...] = cs
    x = x_ref[...]
    xc_ref[0] = x
    xh_ref[0] = dinv * x


def _prologue(deg2, cs2, xp):
    return pl.pallas_call(
        _prologue_body,
        grid=(NB,),
        in_specs=[
            pl.BlockSpec((2, 256, 1), lambda i: (0, i, 0)),
            pl.BlockSpec((2, 256, 1), lambda i: (0, i, 0)),
            pl.BlockSpec((256, 128), lambda i: (i, 0)),
        ],
        out_specs=[
            pl.BlockSpec((256, 1), lambda i: (i, 0)),
            pl.BlockSpec((256, 1), lambda i: (i, 0)),
            pl.BlockSpec((1, 256, 128), lambda i: (0, i, 0)),
            pl.BlockSpec((1, 256, 128), lambda i: (0, i, 0)),
        ],
        out_shape=[
            jax.ShapeDtypeStruct((NP, 1), F32),
            jax.ShapeDtypeStruct((NP, 1), F32),
            jax.ShapeDtypeStruct((1, NP, 128), F32),
            jax.ShapeDtypeStruct((1, NP, 128), F32),
        ],
    )(deg2, cs2, xp)


def _cheb(g, yh, tpp, dinv, cs, alpha):
    if len(g) != len(yh):
        gg = lambda c: g[0] + g[1]
    else:
        gg = lambda c: g[c]
    ts = []
    for c in range(len(yh)):
        t = (-alpha) * dinv * (gg(c) - cs * yh[c])
        if tpp is not None:
            t = t - tpp[c]
        ts.append(t)
    return ts


def _elt_body(nc, alpha, g_ref, yh_ref, tpp_ref, dinv_ref, cs_ref,
              tx_ref, yhn_ref):
    d = dinv_ref[...]
    s = cs_ref[...]
    g = [g_ref[c] for c in range(g_ref.shape[0])]
    yh = [yh_ref[c] for c in range(nc)]
    tpp = None if tpp_ref is yh_ref else [tpp_ref[c] for c in range(nc)]
    ts = _cheb(g, yh, tpp, d, s, alpha)
    for c in range(nc):
        tx_ref[c] = ts[c]
        yhn_ref[c] = d * ts[c]


def _elt(g, yh, tpp, dinv, cs, alpha):
    nc, _, W = yh.shape
    gnc = g.shape[0]

    def body(g_ref, yh_ref, tpp_ref, dinv_ref, cs_ref, tx_ref, yhn_ref):
        _elt_body(nc, alpha, g_ref, yh_ref,
                  yh_ref if tpp is None else tpp_ref,
                  dinv_ref, cs_ref, tx_ref, yhn_ref)

    return pl.pallas_call(
        body,
        grid=(NB,),
        in_specs=[
            pl.BlockSpec((gnc, 256, W), lambda i: (0, i, 0)),
            pl.BlockSpec((nc, 256, W), lambda i: (0, i, 0)),
            pl.BlockSpec((nc, 256, W), lambda i: (0, i, 0)),
            pl.BlockSpec((256, 1), lambda i: (i, 0)),
            pl.BlockSpec((256, 1), lambda i: (i, 0)),
        ],
        out_specs=[
            pl.BlockSpec((nc, 256, W), lambda i: (0, i, 0)),
            pl.BlockSpec((nc, 256, W), lambda i: (0, i, 0)),
        ],
        out_shape=[
            jax.ShapeDtypeStruct((nc, NP, W), F32),
            jax.ShapeDtypeStruct((nc, NP, W), F32),
        ],
    )(g, yh, yh if tpp is None else tpp, dinv, cs)


def _stepf_body(nc, fout, g_ref, yh_ref, tx1_ref, xc_ref, tx2_ref, dinv_ref,
                cs_ref, w_ref, b_ref, acc_ref, sums_ref, sacc_ref):
    i = pl.program_id(0)
    d = dinv_ref[...]
    s = cs_ref[...]
    acc = jnp.broadcast_to(b_ref[...], acc_ref.shape).astype(F32)
    g = [g_ref[c] for c in range(g_ref.shape[0])]
    yh = [yh_ref[c] for c in range(nc)]
    tpp = [tx1_ref[c] for c in range(nc)]
    ts = _cheb(g, yh, tpp, d, s, 2.0)
    for c in range(nc):
        acc = acc + jnp.dot(xc_ref[c], w_ref[0, c], preferred_element_type=F32)
        acc = acc + jnp.dot(tx1_ref[c], w_ref[1, c], preferred_element_type=F32)
        acc = acc + jnp.dot(tx2_ref[c], w_ref[2, c], preferred_element_type=F32)
        acc = acc + jnp.dot(ts[c], w_ref[3, c], preferred_element_type=F32)
    acc_ref[...] = acc
    row = lax.broadcasted_iota(jnp.int32, (256, 1), 0) + i * 256
    am = jnp.where(row < N, acc, F32(0.0))

    @pl.when(i == 0)
    def _():
        sacc_ref[...] = jnp.zeros(sacc_ref.shape, F32)

    sacc_ref[0:1] += jnp.sum(am, axis=0, keepdims=True)
    sacc_ref[1:2] += jnp.sum(am * am, axis=0, keepdims=True)
    sums_ref[...] = sacc_ref[0:2]


def _stepf(g, yh, tx1, xc, tx2, dinv, cs, wc, b):
    nc, _, W = yh.shape
    gnc = g.shape[0]
    fout = wc.shape[3]
    return pl.pallas_call(
        functools.partial(_stepf_body, nc, fout),
        grid=(NB,),
        in_specs=[
            pl.BlockSpec((gnc, 256, W), lambda i: (0, i, 0)),
            pl.BlockSpec((nc, 256, W), lambda i: (0, i, 0)),
            pl.BlockSpec((nc, 256, W), lambda i: (0, i, 0)),
            pl.BlockSpec((nc, 256, W), lambda i: (0, i, 0)),
            pl.BlockSpec((nc, 256, W), lambda i: (0, i, 0)),
            pl.BlockSpec((256, 1), lambda i: (i, 0)),
            pl.BlockSpec((256, 1), lambda i: (i, 0)),
            pl.BlockSpec((4, nc, W, fout), lambda i: (0, 0, 0, 0)),
            pl.BlockSpec((1, fout), lambda i: (0, 0)),
        ],
        out_specs=[
            pl.BlockSpec((256, fout), lambda i: (i, 0)),
            pl.BlockSpec((2, fout), lambda i: (0, 0)),
        ],
        out_shape=[
            jax.ShapeDtypeStruct((NP, fout), F32),
            jax.ShapeDtypeStruct((2, fout), F32),
        ],
        scratch_shapes=[pltpu.VMEM((8, fout), F32)],
    )(g, yh, tx1, xc, tx2, dinv, cs, wc, b.reshape(1, fout))


def _gnorm_y(acc_ref, sums_ref, gw_ref, gb_ref, gm_ref):
    a = acc_ref[...]
    inv_n = F32(1.0 / N)
    mean = sums_ref[0:1] * inv_n
    ex2 = sums_ref[1:2] * inv_n
    mm = mean * gm_ref[...]
    var = ex2 - 2.0 * mm * mean + mm * mm
    std = lax.sqrt(var + F32(1e-5))
    return gw_ref[...] * (a - mm) / std + gb_ref[...]


def _norm_body(fout, ncn, acc_ref, sums_ref, gw_ref, gb_ref, gm_ref,
               dinv_ref, out0_ref, out1_ref):
    y = _gnorm_y(acc_ref, sums_ref, gw_ref, gb_ref, gm_ref)
    y = jnp.where(y >= 0, y, F32(0.1) * y)
    d = dinv_ref[...]
    Wn = fout // ncn
    for c in range(ncn):
        ys = y[:, c * Wn:(c + 1) * Wn]
        out0_ref[c] = ys
        out1_ref[c] = d * ys


def _norm(acc, sums, gw, gb, gm, dinv, ncn):
    fout = acc.shape[1]
    Wn = fout // ncn
    return pl.pallas_call(
        functools.partial(_norm_body, fout, ncn),
        grid=(NB,),
        in_specs=[
            pl.BlockSpec((256, fout), lambda i: (i, 0)),
            pl.BlockSpec((2, fout), lambda i: (0, 0)),
            pl.BlockSpec((1, fout), lambda i: (0, 0)),
            pl.BlockSpec((1, fout), lambda i: (0, 0)),
            pl.BlockSpec((1, fout), lambda i: (0, 0)),
            pl.BlockSpec((256, 1), lambda i: (i, 0)),
        ],
        out_specs=[
            pl.BlockSpec((ncn, 256, Wn), lambda i: (0, i, 0)),
            pl.BlockSpec((ncn, 256, Wn), lambda i: (0, i, 0)),
        ],
        out_shape=[
            jax.ShapeDtypeStruct((ncn, NP, Wn), F32),
            jax.ShapeDtypeStruct((ncn, NP, Wn), F32),
        ],
    )(acc, sums, gw.reshape(1, fout), gb.reshape(1, fout),
      gm.reshape(1, fout), dinv)


def _tail_body(fout, acc_ref, sums_ref, gw_ref, gb_ref, gm_ref, x_ref,
               w1_ref, b1_ref, w2_ref, b2_ref, o_ref, pacc_ref):
    i = pl.program_id(0)
    y = _gnorm_y(acc_ref, sums_ref, gw_ref, gb_ref, gm_ref)
    row = lax.broadcasted_iota(jnp.int32, (256, 1), 0) + i * 256
    h = jnp.maximum(y + x_ref[...], F32(0.0))
    hm = jnp.where(row < N, h, F32(0.0))

    @pl.when(i == 0)
    def _():
        pacc_ref[...] = jnp.zeros(pacc_ref.shape, F32)

    pacc_ref[0:1] += jnp.sum(hm, axis=0, keepdims=True)

    @pl.when(i == NB - 1)
    def _():
        pooled = pacc_ref[0:1] * F32(1.0 / N)
        hh = jnp.tanh(jnp.dot(pooled, w1_ref[...],
                              preferred_element_type=F32) + b1_ref[...])
        o_ref[...] = jnp.dot(hh, w2_ref[...],
                             preferred_element_type=F32) + b2_ref[...]

    @pl.when(i < NB - 1)
    def _():
        o_ref[...] = jnp.zeros(o_ref.shape, F32)


def _tail(acc, sums, gw, gb, gm, xp, l1W, l1b, l2W, l2b):
    fout = acc.shape[1]
    return pl.pallas_call(
        functools.partial(_tail_body, fout),
        grid=(NB,),
        in_specs=[
            pl.BlockSpec((256, fout), lambda i: (i, 0)),
            pl.BlockSpec((2, fout), lambda i: (0, 0)),
            pl.BlockSpec((1, fout), lambda i: (0, 0)),
            pl.BlockSpec((1, fout), lambda i: (0, 0)),
            pl.BlockSpec((1, fout), lambda i: (0, 0)),
            pl.BlockSpec((256, 128), lambda i: (i, 0)),
            pl.BlockSpec((128, 64), lambda i: (0, 0)),
            pl.BlockSpec((1, 64), lambda i: (0, 0)),
            pl.BlockSpec((64, 12), lambda i: (0, 0)),
            pl.BlockSpec((1, 12), lambda i: (0, 0)),
        ],
        out_specs=pl.BlockSpec((1, 12), lambda i: (0, 0)),
        out_shape=jax.ShapeDtypeStruct((1, 12), F32),
        scratch_shapes=[pltpu.VMEM((8, fout), F32)],
    )(acc, sums, gw.reshape(1, fout), gb.reshape(1, fout),
      gm.reshape(1, fout), xp, l1W, l1b.reshape(1, 64), l2W,
      l2b.reshape(1, 12))


_CHUNKS = {128: (1, 128), 256: (2, 128), 512: (4, 128)}


def kernel(x, edge_index, batch, W1, b1, g1w, g1b, g1m, W2, b2, g2w, g2b, g2m,
           W3, b3, g3w, g3b, g3m, W4, b4, g4w, g4b, g4m, l1W, l1b, l2W, l2b):
    del batch
    src = edge_index[0].astype(jnp.int32)
    dst = edge_index[1].astype(jnp.int32)
    pad_idx = (N + jnp.arange(EP - E, dtype=jnp.int32) % (NP - N))
    src_p = jnp.concatenate([src, pad_idx])
    dst_p = jnp.concatenate([dst, pad_idx])
    src16 = src_p.reshape(16, RG, 128)
    dst16 = dst_p.reshape(16, RG, 128)
    src32 = src_p.reshape(32, RD, 128)
    dst32 = dst_p.reshape(32, RD, 128)
    zeros2 = jnp.zeros((NP, 128), F32)
    zeros1 = jnp.zeros((NP,), F32)
    xp = jnp.pad(x, ((0, NP - N), (0, 0)))

    deg2, cs2 = _make_deg_kernel()(src32, dst32, zeros1)
    dinv, csl, xc, xh = _prologue(deg2.reshape(2, NP, 1), cs2.reshape(2, NP, 1), xp)

    layers = [
        (W1, b1, g1w, g1b, g1m),
        (W2, b2, g2w, g2b, g2m),
        (W3, b3, g3w, g3b, g3m),
        (W4, b4, g4w, g4b, g4m),
    ]
    pooled = None
    for li, (Wl, bl, gw, gb, gm) in enumerate(layers):
        K, fin, fout = Wl.shape
        nc, W = _CHUNKS[fin]
        wc = Wl.reshape(K, nc, W, fout)
        g_call = _make_g_kernel(nc)
        e_src = src32 if nc == 1 else src16
        e_dst = dst32 if nc == 1 else dst16
        g1 = g_call(xh, e_src, e_dst, zeros2)
        tx1, yh1 = _elt(g1, xh, None, dinv, csl, 1.0)
        g2 = g_call(yh1, e_src, e_dst, zeros2)
        tx2, yh2 = _elt(g2, yh1, xc, dinv, csl, 2.0)
        g3 = g_call(yh2, e_src, e_dst, zeros2)
        acc, sums = _stepf(g3, yh2, tx1, xc, tx2, dinv, csl, wc, bl)
        if li < 3:
            ncn, Wn = _CHUNKS[fout]
            xc, xh = _norm(acc, sums, gw, gb, gm, dinv, ncn)
        else:
            out = _tail(acc, sums, gw, gb, gm, xp, l1W, l1b, l2W, l2b)

    return out


# speedup vs baseline: 1.0252x; 1.0252x over previous
import functools

import jax
import jax.numpy as jnp
from jax import lax
from jax.experimental import pallas as pl
from jax.experimental.pallas import tpu as pltpu
from jax.experimental.pallas import tpu_sc as plsc

N = 10000
NP = 10240
E = 320000
EP = 327680
RG = 160
RD = 80
NB = NP // 256
ROWS_PER_TILE = NP // 16

F32 = jnp.float32
BF16 = jnp.bfloat16

_SC_MESH = dict(core_axis_name="c", subcore_axis_name="s")


@functools.cache
def _make_g_kernel(nc):
    W = 128
    split_edges = nc == 1
    blocks = [(0, 56), (56, 24)] if split_edges else [(0, 56), (56, 56), (112, 48)]
    RBMAX = 56
    n_out = 2 if split_edges else nc
    chunk_iters = 1 if split_edges else nc // 2

    @functools.partial(
        pl.kernel,
        out_type=jax.ShapeDtypeStruct((n_out, NP, W), F32),
        mesh=plsc.VectorSubcoreMesh(**_SC_MESH),
        cost_estimate=pl.CostEstimate(
            flops=0, transcendentals=0,
            bytes_accessed=nc * EP * W * 4 * 2),
        scratch_types=[
            pltpu.VMEM((RBMAX, 128), jnp.int32),
            pltpu.VMEM((RBMAX, 128), jnp.int32),
            pltpu.VMEM((128, W), F32),
            pltpu.VMEM((128, W), F32),
            pltpu.VMEM_SHARED((NP, W), F32),
            pltpu.SemaphoreType.DMA,
            pltpu.SemaphoreType.DMA,
        ],
    )
    def g_kernel(tab, srcr, dstr, zeros, out, src_v, dst_v, rows0, rows1,
                 accum, sem0, sem1):
        cid = lax.axis_index("c")
        sid = lax.axis_index("s")
        r0 = sid * ROWS_PER_TILE
        my_src = srcr.at[sid * 2 + cid] if split_edges else srcr.at[sid]
        my_dst = dstr.at[sid * 2 + cid] if split_edges else dstr.at[sid]
        for ci in range(chunk_iters):
            c = 0 if split_edges else 2 * ci + cid
            o = cid if split_edges else c
            tab_c = tab.at[c]
            pltpu.sync_copy(zeros.at[pl.ds(r0, ROWS_PER_TILE)],
                            accum.at[pl.ds(r0, ROWS_PER_TILE)])
            plsc.subcore_barrier()

            for off, rb in blocks:
                pltpu.sync_copy(my_src.at[pl.ds(off, rb)],
                                src_v.at[pl.ds(0, rb)])
                pltpu.sync_copy(my_dst.at[pl.ds(off, rb)],
                                dst_v.at[pl.ds(0, rb)])
                pltpu.async_copy(tab_c.at[src_v.at[0]], rows0, sem0)

                def body(i, _, rb=rb):
                    u = 2 * i
                    pltpu.async_copy(tab_c.at[src_v.at[u + 1]], rows1, sem1)
                    pltpu.make_async_copy(tab_c.at[src_v.at[u]], rows0,
                                          sem0).wait()
                    pltpu.sync_copy(rows0, accum.at[dst_v.at[u]], add=True)

                    @pl.when(u + 2 < rb)
                    def _():
                        pltpu.async_copy(tab_c.at[src_v.at[u + 2]], rows0, sem0)

                    pltpu.make_async_copy(tab_c.at[src_v.at[u + 1]], rows1,
                                          sem1).wait()
                    pltpu.sync_copy(rows1, accum.at[dst_v.at[u + 1]], add=True)
                    return 0

                lax.fori_loop(0, rb // 2, body, 0)
            plsc.subcore_barrier()
            pltpu.sync_copy(accum.at[pl.ds(r0, ROWS_PER_TILE)],
                            out.at[o].at[pl.ds(r0, ROWS_PER_TILE)])
            plsc.subcore_barrier()

    return g_kernel


@functools.cache
def _make_deg_kernel():
    @functools.partial(
        pl.kernel,
        out_type=(jax.ShapeDtypeStruct((2, NP), F32),
                  jax.ShapeDtypeStruct((2, NP), F32)),
        mesh=plsc.VectorSubcoreMesh(**_SC_MESH),
        scratch_types=[
            pltpu.VMEM((RD, 128), jnp.int32),
            pltpu.VMEM((RD, 128), jnp.int32),
            pltpu.VMEM((128,), F32),
            pltpu.VMEM((128,), F32),
            pltpu.VMEM_SHARED((NP,), F32),
            pltpu.VMEM_SHARED((NP,), F32),
        ],
    )
    def deg_kernel(srcr, dstr, zeros1, deg_out, cs_out, src_v, dst_v,
                   mval, cval, acc_deg, acc_cs):
        cid = lax.axis_index("c")
        sid = lax.axis_index("s")
        wid = sid * 2 + cid
        r0 = sid * ROWS_PER_TILE
        pltpu.sync_copy(srcr.at[wid], src_v)
        pltpu.sync_copy(dstr.at[wid], dst_v)
        pltpu.sync_copy(zeros1.at[pl.ds(r0, ROWS_PER_TILE)],
                        acc_deg.at[pl.ds(r0, ROWS_PER_TILE)])
        pltpu.sync_copy(zeros1.at[pl.ds(r0, ROWS_PER_TILE)],
                        acc_cs.at[pl.ds(r0, ROWS_PER_TILE)])
        plsc.subcore_barrier()

        def body(j, _):
            for i in range(8):
                s = src_v[j, pl.ds(i * 16, 16)]
                d = dst_v[j, pl.ds(i * 16, 16)]
                m = jnp.where(s != d, F32(1.0), F32(0.0))
                mval[pl.ds(i * 16, 16)] = m
                cval[pl.ds(i * 16, 16)] = F32(1.0) - m
            pltpu.sync_copy(mval, acc_deg.at[src_v.at[j]], add=True)
            pltpu.sync_copy(cval, acc_cs.at[src_v.at[j]], add=True)
            return 0

        lax.fori_loop(0, RD, body, 0)
        plsc.subcore_barrier()
        pltpu.sync_copy(acc_deg.at[pl.ds(r0, ROWS_PER_TILE)],
                        deg_out.at[cid].at[pl.ds(r0, ROWS_PER_TILE)])
        pltpu.sync_copy(acc_cs.at[pl.ds(r0, ROWS_PER_TILE)],
                        cs_out.at[cid].at[pl.ds(r0, ROWS_PER_TILE)])

    return deg_kernel


def _prologue_body(deg2_ref, cs2_ref, x_ref, dinv_ref, cs_ref, xc_ref, xh_ref):
    deg = jnp.sum(deg2_ref[...], axis=0)
    cs = jnp.sum(cs2_ref[...], axis=0)
    dinv = jnp.where(deg > 0, lax.rsqrt(jnp.maximum(deg, F32(1.0))), F32(0.0))
    dinv_ref[...] = dinv
    cs_ref[...] = cs
    x = x_ref[...]
    xc_ref[0] = x
    xh_ref[0] = dinv * x


def _prologue(deg2, cs2, xp):
    return pl.pallas_call(
        _prologue_body,
        grid=(NB,),
        in_specs=[
            pl.BlockSpec((2, 256, 1), lambda i: (0, i, 0)),
            pl.BlockSpec((2, 256, 1), lambda i: (0, i, 0)),
            pl.BlockSpec((256, 128), lambda i: (i, 0)),
        ],
        out_specs=[
            pl.BlockSpec((256, 1), lambda i: (i, 0)),
            pl.BlockSpec((256, 1), lambda i: (i, 0)),
            pl.BlockSpec((1, 256, 128), lambda i: (0, i, 0)),
            pl.BlockSpec((1, 256, 128), lambda i: (0, i, 0)),
        ],
        out_shape=[
            jax.ShapeDtypeStruct((NP, 1), F32),
            jax.ShapeDtypeStruct((NP, 1), F32),
            jax.ShapeDtypeStruct((1, NP, 128), F32),
            jax.ShapeDtypeStruct((1, NP, 128), F32),
        ],
    )(deg2, cs2, xp)


def _cheb(g, yh, tpp, dinv, cs, alpha):
    if len(g) != len(yh):
        gg = lambda c: g[0] + g[1]
    else:
        gg = lambda c: g[c]
    ts = []
    for c in range(len(yh)):
        t = (-alpha) * dinv * (gg(c) - cs * yh[c])
        if tpp is not None:
            t = t - tpp[c]
        ts.append(t)
    return ts


def _elt_body(nc, alpha, g_ref, yh_ref, tpp_ref, dinv_ref, cs_ref,
              tx_ref, yhn_ref):
    d = dinv_ref[...]
    s = cs_ref[...]
    g = [g_ref[c] for c in range(g_ref.shape[0])]
    yh = [yh_ref[c] for c in range(nc)]
    tpp = None if tpp_ref is yh_ref else [tpp_ref[c] for c in range(nc)]
    ts = _cheb(g, yh, tpp, d, s, alpha)
    for c in range(nc):
        tx_ref[c] = ts[c]
        yhn_ref[c] = d * ts[c]


def _elt(g, yh, tpp, dinv, cs, alpha):
    nc, _, W = yh.shape
    gnc = g.shape[0]

    def body(g_ref, yh_ref, tpp_ref, dinv_ref, cs_ref, tx_ref, yhn_ref):
        _elt_body(nc, alpha, g_ref, yh_ref,
                  yh_ref if tpp is None else tpp_ref,
                  dinv_ref, cs_ref, tx_ref, yhn_ref)

    return pl.pallas_call(
        body,
        grid=(NB,),
        in_specs=[
            pl.BlockSpec((gnc, 256, W), lambda i: (0, i, 0)),
            pl.BlockSpec((nc, 256, W), lambda i: (0, i, 0)),
            pl.BlockSpec((nc, 256, W), lambda i: (0, i, 0)),
            pl.BlockSpec((256, 1), lambda i: (i, 0)),
            pl.BlockSpec((256, 1), lambda i: (i, 0)),
        ],
        out_specs=[
            pl.BlockSpec((nc, 256, W), lambda i: (0, i, 0)),
            pl.BlockSpec((nc, 256, W), lambda i: (0, i, 0)),
        ],
        out_shape=[
            jax.ShapeDtypeStruct((nc, NP, W), F32),
            jax.ShapeDtypeStruct((nc, NP, W), F32),
        ],
    )(g, yh, yh if tpp is None else tpp, dinv, cs)


def _stepf_body(nc, fout, g_ref, yh_ref, tx1_ref, xc_ref, tx2_ref, dinv_ref,
                cs_ref, w_ref, b_ref, acc_ref, sums_ref, sacc_ref):
    i = pl.program_id(0)
    d = dinv_ref[...]
    s = cs_ref[...]
    acc = jnp.broadcast_to(b_ref[...], acc_ref.shape).astype(F32)
    g = [g_ref[c] for c in range(g_ref.shape[0])]
    yh = [yh_ref[c] for c in range(nc)]
    tpp = [tx1_ref[c] for c in range(nc)]
    ts = _cheb(g, yh, tpp, d, s, 2.0)
    for c in range(nc):
        acc = acc + jnp.dot(xc_ref[c], w_ref[0, c], preferred_element_type=F32)
        acc = acc + jnp.dot(tx1_ref[c], w_ref[1, c], preferred_element_type=F32)
        acc = acc + jnp.dot(tx2_ref[c], w_ref[2, c], preferred_element_type=F32)
        acc = acc + jnp.dot(ts[c], w_ref[3, c], preferred_element_type=F32)
    acc_ref[...] = acc
    row = lax.broadcasted_iota(jnp.int32, (256, 1), 0) + i * 256
    am = jnp.where(row < N, acc, F32(0.0))

    @pl.when(i == 0)
    def _():
        sacc_ref[...] = jnp.zeros(sacc_ref.shape, F32)

    sacc_ref[0:1] += jnp.sum(am, axis=0, keepdims=True)
    sacc_ref[1:2] += jnp.sum(am * am, axis=0, keepdims=True)
    sums_ref[...] = sacc_ref[0:2]


def _stepf(g, yh, tx1, xc, tx2, dinv, cs, wc, b):
    nc, _, W = yh.shape
    gnc = g.shape[0]
    fout = wc.shape[3]
    return pl.pallas_call(
        functools.partial(_stepf_body, nc, fout),
        grid=(NB,),
        in_specs=[
            pl.BlockSpec((gnc, 256, W), lambda i: (0, i, 0)),
            pl.BlockSpec((nc, 256, W), lambda i: (0, i, 0)),
            pl.BlockSpec((nc, 256, W), lambda i: (0, i, 0)),
            pl.BlockSpec((nc, 256, W), lambda i: (0, i, 0)),
            pl.BlockSpec((nc, 256, W), lambda i: (0, i, 0)),
            pl.BlockSpec((256, 1), lambda i: (i, 0)),
            pl.BlockSpec((256, 1), lambda i: (i, 0)),
            pl.BlockSpec((4, nc, W, fout), lambda i: (0, 0, 0, 0)),
            pl.BlockSpec((1, fout), lambda i: (0, 0)),
        ],
        out_specs=[
            pl.BlockSpec((256, fout), lambda i: (i, 0)),
            pl.BlockSpec((2, fout), lambda i: (0, 0)),
        ],
        out_shape=[
            jax.ShapeDtypeStruct((NP, fout), F32),
            jax.ShapeDtypeStruct((2, fout), F32),
        ],
        scratch_shapes=[pltpu.VMEM((8, fout), F32)],
    )(g, yh, tx1, xc, tx2, dinv, cs, wc, b.reshape(1, fout))


def _gnorm_y(acc_ref, sums_ref, gw_ref, gb_ref, gm_ref):
    a = acc_ref[...]
    inv_n = F32(1.0 / N)
    mean = sums_ref[0:1] * inv_n
    ex2 = sums_ref[1:2] * inv_n
    mm = mean * gm_ref[...]
    var = ex2 - 2.0 * mm * mean + mm * mm
    std = lax.sqrt(var + F32(1e-5))
    return gw_ref[...] * (a - mm) / std + gb_ref[...]


def _norm_body(fout, ncn, acc_ref, sums_ref, gw_ref, gb_ref, gm_ref,
               dinv_ref, out0_ref, out1_ref):
    y = _gnorm_y(acc_ref, sums_ref, gw_ref, gb_ref, gm_ref)
    y = jnp.where(y >= 0, y, F32(0.1) * y)
    d = dinv_ref[...]
    Wn = fout // ncn
    for c in range(ncn):
        ys = y[:, c * Wn:(c + 1) * Wn]
        out0_ref[c] = ys
        out1_ref[c] = d * ys


def _norm(acc, sums, gw, gb, gm, dinv, ncn):
    fout = acc.shape[1]
    Wn = fout // ncn
    return pl.pallas_call(
        functools.partial(_norm_body, fout, ncn),
        grid=(NB,),
        in_specs=[
            pl.BlockSpec((256, fout), lambda i: (i, 0)),
            pl.BlockSpec((2, fout), lambda i: (0, 0)),
            pl.BlockSpec((1, fout), lambda i: (0, 0)),
            pl.BlockSpec((1, fout), lambda i: (0, 0)),
            pl.BlockSpec((1, fout), lambda i: (0, 0)),
            pl.BlockSpec((256, 1), lambda i: (i, 0)),
        ],
        out_specs=[
            pl.BlockSpec((ncn, 256, Wn), lambda i: (0, i, 0)),
            pl.BlockSpec((ncn, 256, Wn), lambda i: (0, i, 0)),
        ],
        out_shape=[
            jax.ShapeDtypeStruct((ncn, NP, Wn), F32),
            jax.ShapeDtypeStruct((ncn, NP, Wn), F32),
        ],
    )(acc, sums, gw.reshape(1, fout), gb.reshape(1, fout),
      gm.reshape(1, fout), dinv)


def _tail_body(fout, acc_ref, sums_ref, gw_ref, gb_ref, gm_ref, x_ref,
               w1_ref, b1_ref, w2_ref, b2_ref, o_ref, pacc_ref):
    i = pl.program_id(0)
    y = _gnorm_y(acc_ref, sums_ref, gw_ref, gb_ref, gm_ref)
    row = lax.broadcasted_iota(jnp.int32, (256, 1), 0) + i * 256
    h = jnp.maximum(y + x_ref[...], F32(0.0))
    hm = jnp.where(row < N, h, F32(0.0))

    @pl.when(i == 0)
    def _():
        pacc_ref[...] = jnp.zeros(pacc_ref.shape, F32)

    pacc_ref[0:1] += jnp.sum(hm, axis=0, keepdims=True)

    @pl.when(i == NB - 1)
    def _():
        pooled = pacc_ref[0:1] * F32(1.0 / N)
        hh = jnp.tanh(jnp.dot(pooled, w1_ref[...],
                              preferred_element_type=F32) + b1_ref[...])
        o_ref[...] = jnp.dot(hh, w2_ref[...],
                             preferred_element_type=F32) + b2_ref[...]

    @pl.when(i < NB - 1)
    def _():
        o_ref[...] = jnp.zeros(o_ref.shape, F32)


def _tail(acc, sums, gw, gb, gm, xp, l1W, l1b, l2W, l2b):
    fout = acc.shape[1]
    return pl.pallas_call(
        functools.partial(_tail_body, fout),
        grid=(NB,),
        in_specs=[
            pl.BlockSpec((256, fout), lambda i: (i, 0)),
            pl.BlockSpec((2, fout), lambda i: (0, 0)),
            pl.BlockSpec((1, fout), lambda i: (0, 0)),
            pl.BlockSpec((1, fout), lambda i: (0, 0)),
            pl.BlockSpec((1, fout), lambda i: (0, 0)),
            pl.BlockSpec((256, 128), lambda i: (i, 0)),
            pl.BlockSpec((128, 64), lambda i: (0, 0)),
            pl.BlockSpec((1, 64), lambda i: (0, 0)),
            pl.BlockSpec((64, 12), lambda i: (0, 0)),
            pl.BlockSpec((1, 12), lambda i: (0, 0)),
        ],
        out_specs=pl.BlockSpec((1, 12), lambda i: (0, 0)),
        out_shape=jax.ShapeDtypeStruct((1, 12), F32),
        scratch_shapes=[pltpu.VMEM((8, fout), F32)],
    )(acc, sums, gw.reshape(1, fout), gb.reshape(1, fout),
      gm.reshape(1, fout), xp, l1W, l1b.reshape(1, 64), l2W,
      l2b.reshape(1, 12))


_CHUNKS = {128: (1, 128), 256: (2, 128), 512: (4, 128)}


def kernel(x, edge_index, batch, W1, b1, g1w, g1b, g1m, W2, b2, g2w, g2b, g2m,
           W3, b3, g3w, g3b, g3m, W4, b4, g4w, g4b, g4m, l1W, l1b, l2W, l2b):
    del batch
    src = edge_index[0].astype(jnp.int32)
    dst = edge_index[1].astype(jnp.int32)
    pad_idx = (N + jnp.arange(EP - E, dtype=jnp.int32) % (NP - N))
    src_p = jnp.concatenate([src, pad_idx])
    dst_p = jnp.concatenate([dst, pad_idx])
    src16 = src_p.reshape(16, RG, 128)
    dst16 = dst_p.reshape(16, RG, 128)
    src32 = src_p.reshape(32, RD, 128)
    dst32 = dst_p.reshape(32, RD, 128)
    zeros2 = jnp.zeros((NP, 128), F32)
    zeros1 = jnp.zeros((NP,), F32)
    xp = jnp.pad(x, ((0, NP - N), (0, 0)))

    deg2, cs2 = _make_deg_kernel()(src32, dst32, zeros1)
    dinv, csl, xc, xh = _prologue(deg2.reshape(2, NP, 1), cs2.reshape(2, NP, 1), xp)

    layers = [
        (W1, b1, g1w, g1b, g1m),
        (W2, b2, g2w, g2b, g2m),
        (W3, b3, g3w, g3b, g3m),
        (W4, b4, g4w, g4b, g4m),
    ]
    pooled = None
    for li, (Wl, bl, gw, gb, gm) in enumerate(layers):
        K, fin, fout = Wl.shape
        nc, W = _CHUNKS[fin]
        wc = Wl.reshape(K, nc, W, fout)
        g_call = _make_g_kernel(nc)
        e_src = src32 if nc == 1 else src16
        e_dst = dst32 if nc == 1 else dst16
        g1 = g_call(xh, e_src, e_dst, zeros2)
        tx1, yh1 = _elt(g1, xh, None, dinv, csl, 1.0)
        g2 = g_call(yh1, e_src, e_dst, zeros2)
        tx2, yh2 = _elt(g2, yh1, xc, dinv, csl, 2.0)
        g3 = g_call(yh2, e_src, e_dst, zeros2)
        acc, sums = _stepf(g3, yh2, tx1, xc, tx2, dinv, csl, wc, bl)
        if li < 3:
            ncn, Wn = _CHUNKS[fout]
            xc, xh = _norm(acc, sums, gw, gb, gm, dinv, ncn)
        else:
            out = _tail(acc, sums, gw, gb, gm, xp, l1W, l1b, l2W, l2b)

    return out


# speedup vs baseline: 1.0288x; 1.0035x over previous
import functools

import jax
import jax.numpy as jnp
from jax import lax
from jax.experimental import pallas as pl
from jax.experimental.pallas import tpu as pltpu
from jax.experimental.pallas import tpu_sc as plsc

N = 10000
NP = 10240
E = 320000
EP = 327680
RG = 160
RD = 80
NB = NP // 256
ROWS_PER_TILE = NP // 16

F32 = jnp.float32
BF16 = jnp.bfloat16

_SC_MESH = dict(core_axis_name="c", subcore_axis_name="s")


@functools.cache
def _make_g_kernel(nc):
    W = 128
    split_edges = nc == 1
    blocks = [(0, 56), (56, 24)] if split_edges else [(0, 56), (56, 56), (112, 48)]
    RBMAX = 56
    n_out = 2 if split_edges else nc
    chunk_iters = 1 if split_edges else nc // 2

    @functools.partial(
        pl.kernel,
        out_type=jax.ShapeDtypeStruct((n_out, NP, W), F32),
        mesh=plsc.VectorSubcoreMesh(**_SC_MESH),
        cost_estimate=pl.CostEstimate(
            flops=0, transcendentals=0,
            bytes_accessed=nc * EP * W * 4 * 2),
        scratch_types=[
            pltpu.VMEM((RBMAX, 128), jnp.int32),
            pltpu.VMEM((RBMAX, 128), jnp.int32),
            pltpu.VMEM((128, W), F32),
            pltpu.VMEM((128, W), F32),
            pltpu.VMEM_SHARED((NP, W), F32),
            pltpu.SemaphoreType.DMA,
            pltpu.SemaphoreType.DMA,
        ],
    )
    def g_kernel(tab, srcr, dstr, zeros, out, src_v, dst_v, rows0, rows1,
                 accum, sem0, sem1):
        cid = lax.axis_index("c")
        sid = lax.axis_index("s")
        r0 = sid * ROWS_PER_TILE
        my_src = srcr.at[sid * 2 + cid] if split_edges else srcr.at[sid]
        my_dst = dstr.at[sid * 2 + cid] if split_edges else dstr.at[sid]
        for ci in range(chunk_iters):
            c = 0 if split_edges else 2 * ci + cid
            o = cid if split_edges else c
            tab_c = tab.at[c]
            pltpu.sync_copy(zeros.at[pl.ds(r0, ROWS_PER_TILE)],
                            accum.at[pl.ds(r0, ROWS_PER_TILE)])
            plsc.subcore_barrier()

            for off, rb in blocks:
                pltpu.sync_copy(my_src.at[pl.ds(off, rb)],
                                src_v.at[pl.ds(0, rb)])
                pltpu.sync_copy(my_dst.at[pl.ds(off, rb)],
                                dst_v.at[pl.ds(0, rb)])
                pltpu.async_copy(tab_c.at[src_v.at[0]], rows0, sem0)

                def body(i, _, rb=rb):
                    u = 2 * i
                    pltpu.async_copy(tab_c.at[src_v.at[u + 1]], rows1, sem1)
                    pltpu.make_async_copy(tab_c.at[src_v.at[u]], rows0,
                                          sem0).wait()
                    pltpu.sync_copy(rows0, accum.at[dst_v.at[u]], add=True)

                    @pl.when(u + 2 < rb)
                    def _():
                        pltpu.async_copy(tab_c.at[src_v.at[u + 2]], rows0, sem0)

                    pltpu.make_async_copy(tab_c.at[src_v.at[u + 1]], rows1,
                                          sem1).wait()
                    pltpu.sync_copy(rows1, accum.at[dst_v.at[u + 1]], add=True)
                    return 0

                lax.fori_loop(0, rb // 2, body, 0)
            plsc.subcore_barrier()
            pltpu.sync_copy(accum.at[pl.ds(r0, ROWS_PER_TILE)],
                            out.at[o].at[pl.ds(r0, ROWS_PER_TILE)])
            plsc.subcore_barrier()

    return g_kernel


@functools.cache
def _make_deg_kernel():
    @functools.partial(
        pl.kernel,
        out_type=(jax.ShapeDtypeStruct((2, NP), F32),
                  jax.ShapeDtypeStruct((2, NP), F32)),
        mesh=plsc.VectorSubcoreMesh(**_SC_MESH),
        scratch_types=[
            pltpu.VMEM((RD, 128), jnp.int32),
            pltpu.VMEM((RD, 128), jnp.int32),
            pltpu.VMEM((RD, 128), F32),
            pltpu.VMEM((RD, 128), F32),
            pltpu.VMEM_SHARED((NP,), F32),
            pltpu.VMEM_SHARED((NP,), F32),
        ],
    )
    def deg_kernel(srcr, dstr, zeros1, deg_out, cs_out, src_v, dst_v,
                   mval, cval, acc_deg, acc_cs):
        cid = lax.axis_index("c")
        sid = lax.axis_index("s")
        wid = sid * 2 + cid
        r0 = sid * ROWS_PER_TILE
        pltpu.sync_copy(srcr.at[wid], src_v)
        pltpu.sync_copy(dstr.at[wid], dst_v)
        pltpu.sync_copy(zeros1.at[pl.ds(r0, ROWS_PER_TILE)],
                        acc_deg.at[pl.ds(r0, ROWS_PER_TILE)])
        pltpu.sync_copy(zeros1.at[pl.ds(r0, ROWS_PER_TILE)],
                        acc_cs.at[pl.ds(r0, ROWS_PER_TILE)])
        plsc.subcore_barrier()

        def body(j, _):
            for i in range(8):
                s = src_v[j, pl.ds(i * 16, 16)]
                d = dst_v[j, pl.ds(i * 16, 16)]
                m = jnp.where(s != d, F32(1.0), F32(0.0))
                mval[j, pl.ds(i * 16, 16)] = m
                cval[j, pl.ds(i * 16, 16)] = F32(1.0) - m
            pltpu.sync_copy(mval.at[j], acc_deg.at[src_v.at[j]], add=True)
            pltpu.sync_copy(cval.at[j], acc_cs.at[src_v.at[j]], add=True)
            return 0

        lax.fori_loop(0, RD, body, 0)
        plsc.subcore_barrier()
        pltpu.sync_copy(acc_deg.at[pl.ds(r0, ROWS_PER_TILE)],
                        deg_out.at[cid].at[pl.ds(r0, ROWS_PER_TILE)])
        pltpu.sync_copy(acc_cs.at[pl.ds(r0, ROWS_PER_TILE)],
                        cs_out.at[cid].at[pl.ds(r0, ROWS_PER_TILE)])

    return deg_kernel


def _prologue_body(deg2_ref, cs2_ref, x_ref, dinv_ref, cs_ref, xc_ref, xh_ref):
    deg = jnp.sum(deg2_ref[...], axis=0)
    cs = jnp.sum(cs2_ref[...], axis=0)
    dinv = jnp.where(deg > 0, lax.rsqrt(jnp.maximum(deg, F32(1.0))), F32(0.0))
    dinv_ref[...] = dinv
    cs_ref[...] = cs
    x = x_ref[...]
    xc_ref[0] = x
    xh_ref[0] = dinv * x


def _prologue(deg2, cs2, xp):
    return pl.pallas_call(
        _prologue_body,
        grid=(NB,),
        in_specs=[
            pl.BlockSpec((2, 256, 1), lambda i: (0, i, 0)),
            pl.BlockSpec((2, 256, 1), lambda i: (0, i, 0)),
            pl.BlockSpec((256, 128), lambda i: (i, 0)),
        ],
        out_specs=[
            pl.BlockSpec((256, 1), lambda i: (i, 0)),
            pl.BlockSpec((256, 1), lambda i: (i, 0)),
            pl.BlockSpec((1, 256, 128), lambda i: (0, i, 0)),
            pl.BlockSpec((1, 256, 128), lambda i: (0, i, 0)),
        ],
        out_shape=[
            jax.ShapeDtypeStruct((NP, 1), F32),
            jax.ShapeDtypeStruct((NP, 1), F32),
            jax.ShapeDtypeStruct((1, NP, 128), F32),
            jax.ShapeDtypeStruct((1, NP, 128), F32),
        ],
    )(deg2, cs2, xp)


def _cheb(g, yh, tpp, dinv, cs, alpha):
    if len(g) != len(yh):
        gg = lambda c: g[0] + g[1]
    else:
        gg = lambda c: g[c]
    ts = []
    for c in range(len(yh)):
        t = (-alpha) * dinv * (gg(c) - cs * yh[c])
        if tpp is not None:
            t = t - tpp[c]
        ts.append(t)
    return ts


def _elt_body(nc, alpha, g_ref, yh_ref, tpp_ref, dinv_ref, cs_ref,
              tx_ref, yhn_ref):
    d = dinv_ref[...]
    s = cs_ref[...]
    g = [g_ref[c] for c in range(g_ref.shape[0])]
    yh = [yh_ref[c] for c in range(nc)]
    tpp = None if tpp_ref is yh_ref else [tpp_ref[c] for c in range(nc)]
    ts = _cheb(g, yh, tpp, d, s, alpha)
    for c in range(nc):
        tx_ref[c] = ts[c]
        yhn_ref[c] = d * ts[c]


def _elt(g, yh, tpp, dinv, cs, alpha):
    nc, _, W = yh.shape
    gnc = g.shape[0]

    def body(g_ref, yh_ref, tpp_ref, dinv_ref, cs_ref, tx_ref, yhn_ref):
        _elt_body(nc, alpha, g_ref, yh_ref,
                  yh_ref if tpp is None else tpp_ref,
                  dinv_ref, cs_ref, tx_ref, yhn_ref)

    return pl.pallas_call(
        body,
        grid=(NB,),
        in_specs=[
            pl.BlockSpec((gnc, 256, W), lambda i: (0, i, 0)),
            pl.BlockSpec((nc, 256, W), lambda i: (0, i, 0)),
            pl.BlockSpec((nc, 256, W), lambda i: (0, i, 0)),
            pl.BlockSpec((256, 1), lambda i: (i, 0)),
            pl.BlockSpec((256, 1), lambda i: (i, 0)),
        ],
        out_specs=[
            pl.BlockSpec((nc, 256, W), lambda i: (0, i, 0)),
            pl.BlockSpec((nc, 256, W), lambda i: (0, i, 0)),
        ],
        out_shape=[
            jax.ShapeDtypeStruct((nc, NP, W), F32),
            jax.ShapeDtypeStruct((nc, NP, W), F32),
        ],
    )(g, yh, yh if tpp is None else tpp, dinv, cs)


def _stepf_body(nc, fout, g_ref, yh_ref, tx1_ref, xc_ref, tx2_ref, dinv_ref,
                cs_ref, w_ref, b_ref, acc_ref, sums_ref, sacc_ref):
    i = pl.program_id(0)
    d = dinv_ref[...]
    s = cs_ref[...]
    acc = jnp.broadcast_to(b_ref[...], acc_ref.shape).astype(F32)
    g = [g_ref[c] for c in range(g_ref.shape[0])]
    yh = [yh_ref[c] for c in range(nc)]
    tpp = [tx1_ref[c] for c in range(nc)]
    ts = _cheb(g, yh, tpp, d, s, 2.0)
    for c in range(nc):
        acc = acc + jnp.dot(xc_ref[c], w_ref[0, c], preferred_element_type=F32)
        acc = acc + jnp.dot(tx1_ref[c], w_ref[1, c], preferred_element_type=F32)
        acc = acc + jnp.dot(tx2_ref[c], w_ref[2, c], preferred_element_type=F32)
        acc = acc + jnp.dot(ts[c], w_ref[3, c], preferred_element_type=F32)
    acc_ref[...] = acc
    row = lax.broadcasted_iota(jnp.int32, (256, 1), 0) + i * 256
    am = jnp.where(row < N, acc, F32(0.0))

    @pl.when(i == 0)
    def _():
        sacc_ref[...] = jnp.zeros(sacc_ref.shape, F32)

    sacc_ref[0:1] += jnp.sum(am, axis=0, keepdims=True)
    sacc_ref[1:2] += jnp.sum(am * am, axis=0, keepdims=True)
    sums_ref[...] = sacc_ref[0:2]


def _stepf(g, yh, tx1, xc, tx2, dinv, cs, wc, b):
    nc, _, W = yh.shape
    gnc = g.shape[0]
    fout = wc.shape[3]
    return pl.pallas_call(
        functools.partial(_stepf_body, nc, fout),
        grid=(NB,),
        in_specs=[
            pl.BlockSpec((gnc, 256, W), lambda i: (0, i, 0)),
            pl.BlockSpec((nc, 256, W), lambda i: (0, i, 0)),
            pl.BlockSpec((nc, 256, W), lambda i: (0, i, 0)),
            pl.BlockSpec((nc, 256, W), lambda i: (0, i, 0)),
            pl.BlockSpec((nc, 256, W), lambda i: (0, i, 0)),
            pl.BlockSpec((256, 1), lambda i: (i, 0)),
            pl.BlockSpec((256, 1), lambda i: (i, 0)),
            pl.BlockSpec((4, nc, W, fout), lambda i: (0, 0, 0, 0)),
            pl.BlockSpec((1, fout), lambda i: (0, 0)),
        ],
        out_specs=[
            pl.BlockSpec((256, fout), lambda i: (i, 0)),
            pl.BlockSpec((2, fout), lambda i: (0, 0)),
        ],
        out_shape=[
            jax.ShapeDtypeStruct((NP, fout), F32),
            jax.ShapeDtypeStruct((2, fout), F32),
        ],
        scratch_shapes=[pltpu.VMEM((8, fout), F32)],
    )(g, yh, tx1, xc, tx2, dinv, cs, wc, b.reshape(1, fout))


def _gnorm_y(acc_ref, sums_ref, gw_ref, gb_ref, gm_ref):
    a = acc_ref[...]
    inv_n = F32(1.0 / N)
    mean = sums_ref[0:1] * inv_n
    ex2 = sums_ref[1:2] * inv_n
    mm = mean * gm_ref[...]
    var = ex2 - 2.0 * mm * mean + mm * mm
    std = lax.sqrt(var + F32(1e-5))
    return gw_ref[...] * (a - mm) / std + gb_ref[...]


def _norm_body(fout, ncn, acc_ref, sums_ref, gw_ref, gb_ref, gm_ref,
               dinv_ref, out0_ref, out1_ref):
    y = _gnorm_y(acc_ref, sums_ref, gw_ref, gb_ref, gm_ref)
    y = jnp.where(y >= 0, y, F32(0.1) * y)
    d = dinv_ref[...]
    Wn = fout // ncn
    for c in range(ncn):
        ys = y[:, c * Wn:(c + 1) * Wn]
        out0_ref[c] = ys
        out1_ref[c] = d * ys


def _norm(acc, sums, gw, gb, gm, dinv, ncn):
    fout = acc.shape[1]
    Wn = fout // ncn
    return pl.pallas_call(
        functools.partial(_norm_body, fout, ncn),
        grid=(NB,),
        in_specs=[
            pl.BlockSpec((256, fout), lambda i: (i, 0)),
            pl.BlockSpec((2, fout), lambda i: (0, 0)),
            pl.BlockSpec((1, fout), lambda i: (0, 0)),
            pl.BlockSpec((1, fout), lambda i: (0, 0)),
            pl.BlockSpec((1, fout), lambda i: (0, 0)),
            pl.BlockSpec((256, 1), lambda i: (i, 0)),
        ],
        out_specs=[
            pl.BlockSpec((ncn, 256, Wn), lambda i: (0, i, 0)),
            pl.BlockSpec((ncn, 256, Wn), lambda i: (0, i, 0)),
        ],
        out_shape=[
            jax.ShapeDtypeStruct((ncn, NP, Wn), F32),
            jax.ShapeDtypeStruct((ncn, NP, Wn), F32),
        ],
    )(acc, sums, gw.reshape(1, fout), gb.reshape(1, fout),
      gm.reshape(1, fout), dinv)


def _tail_body(fout, acc_ref, sums_ref, gw_ref, gb_ref, gm_ref, x_ref,
               w1_ref, b1_ref, w2_ref, b2_ref, o_ref, pacc_ref):
    i = pl.program_id(0)
    y = _gnorm_y(acc_ref, sums_ref, gw_ref, gb_ref, gm_ref)
    row = lax.broadcasted_iota(jnp.int32, (256, 1), 0) + i * 256
    h = jnp.maximum(y + x_ref[...], F32(0.0))
    hm = jnp.where(row < N, h, F32(0.0))

    @pl.when(i == 0)
    def _():
        pacc_ref[...] = jnp.zeros(pacc_ref.shape, F32)

    pacc_ref[0:1] += jnp.sum(hm, axis=0, keepdims=True)

    @pl.when(i == NB - 1)
    def _():
        pooled = pacc_ref[0:1] * F32(1.0 / N)
        hh = jnp.tanh(jnp.dot(pooled, w1_ref[...],
                              preferred_element_type=F32) + b1_ref[...])
        o_ref[...] = jnp.dot(hh, w2_ref[...],
                             preferred_element_type=F32) + b2_ref[...]

    @pl.when(i < NB - 1)
    def _():
        o_ref[...] = jnp.zeros(o_ref.shape, F32)


def _tail(acc, sums, gw, gb, gm, xp, l1W, l1b, l2W, l2b):
    fout = acc.shape[1]
    return pl.pallas_call(
        functools.partial(_tail_body, fout),
        grid=(NB,),
        in_specs=[
            pl.BlockSpec((256, fout), lambda i: (i, 0)),
            pl.BlockSpec((2, fout), lambda i: (0, 0)),
            pl.BlockSpec((1, fout), lambda i: (0, 0)),
            pl.BlockSpec((1, fout), lambda i: (0, 0)),
            pl.BlockSpec((1, fout), lambda i: (0, 0)),
            pl.BlockSpec((256, 128), lambda i: (i, 0)),
            pl.BlockSpec((128, 64), lambda i: (0, 0)),
            pl.BlockSpec((1, 64), lambda i: (0, 0)),
            pl.BlockSpec((64, 12), lambda i: (0, 0)),
            pl.BlockSpec((1, 12), lambda i: (0, 0)),
        ],
        out_specs=pl.BlockSpec((1, 12), lambda i: (0, 0)),
        out_shape=jax.ShapeDtypeStruct((1, 12), F32),
        scratch_shapes=[pltpu.VMEM((8, fout), F32)],
    )(acc, sums, gw.reshape(1, fout), gb.reshape(1, fout),
      gm.reshape(1, fout), xp, l1W, l1b.reshape(1, 64), l2W,
      l2b.reshape(1, 12))


_CHUNKS = {128: (1, 128), 256: (2, 128), 512: (4, 128)}


def kernel(x, edge_index, batch, W1, b1, g1w, g1b, g1m, W2, b2, g2w, g2b, g2m,
           W3, b3, g3w, g3b, g3m, W4, b4, g4w, g4b, g4m, l1W, l1b, l2W, l2b):
    del batch
    src = edge_index[0].astype(jnp.int32)
    dst = edge_index[1].astype(jnp.int32)
    pad_idx = (N + jnp.arange(EP - E, dtype=jnp.int32) % (NP - N))
    src_p = jnp.concatenate([src, pad_idx])
    dst_p = jnp.concatenate([dst, pad_idx])
    src16 = src_p.reshape(16, RG, 128)
    dst16 = dst_p.reshape(16, RG, 128)
    src32 = src_p.reshape(32, RD, 128)
    dst32 = dst_p.reshape(32, RD, 128)
    zeros2 = jnp.zeros((NP, 128), F32)
    zeros1 = jnp.zeros((NP,), F32)
    xp = jnp.pad(x, ((0, NP - N), (0, 0)))

    deg2, cs2 = _make_deg_kernel()(src32, dst32, zeros1)
    dinv, csl, xc, xh = _prologue(deg2.reshape(2, NP, 1), cs2.reshape(2, NP, 1), xp)

    layers = [
        (W1, b1, g1w, g1b, g1m),
        (W2, b2, g2w, g2b, g2m),
        (W3, b3, g3w, g3b, g3m),
        (W4, b4, g4w, g4b, g4m),
    ]
    pooled = None
    for li, (Wl, bl, gw, gb, gm) in enumerate(layers):
        K, fin, fout = Wl.shape
        nc, W = _CHUNKS[fin]
        wc = Wl.reshape(K, nc, W, fout)
        g_call = _make_g_kernel(nc)
        e_src = src32 if nc == 1 else src16
        e_dst = dst32 if nc == 1 else dst16
        g1 = g_call(xh, e_src, e_dst, zeros2)
        tx1, yh1 = _elt(g1, xh, None, dinv, csl, 1.0)
        g2 = g_call(yh1, e_src, e_dst, zeros2)
        tx2, yh2 = _elt(g2, yh1, xc, dinv, csl, 2.0)
        g3 = g_call(yh2, e_src, e_dst, zeros2)
        acc, sums = _stepf(g3, yh2, tx1, xc, tx2, dinv, csl, wc, bl)
        if li < 3:
            ncn, Wn = _CHUNKS[fout]
            xc, xh = _norm(acc, sums, gw, gb, gm, dinv, ncn)
        else:
            out = _tail(acc, sums, gw, gb, gm, xp, l1W, l1b, l2W, l2b)

    return out


# speedup vs baseline: 1.0301x; 1.0012x over previous
import functools

import jax
import jax.numpy as jnp
from jax import lax
from jax.experimental import pallas as pl
from jax.experimental.pallas import tpu as pltpu
from jax.experimental.pallas import tpu_sc as plsc

N = 10000
NP = 10240
E = 320000
EP = 327680
RG = 160
RD = 80
NB = NP // 256
ROWS_PER_TILE = NP // 16

F32 = jnp.float32

_SC_MESH = dict(core_axis_name="c", subcore_axis_name="s")


@functools.cache
def _make_g_kernel(nc):
    W = 128
    split_edges = nc == 1
    blocks = [(0, 56), (56, 24)] if split_edges else [(0, 56), (56, 56), (112, 48)]
    RBMAX = 56
    n_out = 2 if split_edges else nc
    chunk_iters = 1 if split_edges else nc // 2

    @functools.partial(
        pl.kernel,
        out_type=jax.ShapeDtypeStruct((n_out, NP, W), F32),
        mesh=plsc.VectorSubcoreMesh(**_SC_MESH),
        cost_estimate=pl.CostEstimate(
            flops=0, transcendentals=0,
            bytes_accessed=nc * EP * W * 4 * 2),
        scratch_types=[
            pltpu.VMEM((RBMAX, 128), jnp.int32),
            pltpu.VMEM((RBMAX, 128), jnp.int32),
            pltpu.VMEM((128, W), F32),
            pltpu.VMEM((128, W), F32),
            pltpu.VMEM_SHARED((NP, W), F32),
            pltpu.SemaphoreType.DMA,
            pltpu.SemaphoreType.DMA,
        ],
    )
    def g_kernel(tab, srcr, dstr, zeros, out, src_v, dst_v, rows0, rows1,
                 accum, sem0, sem1):
        cid = lax.axis_index("c")
        sid = lax.axis_index("s")
        r0 = sid * ROWS_PER_TILE
        my_src = srcr.at[sid * 2 + cid] if split_edges else srcr.at[sid]
        my_dst = dstr.at[sid * 2 + cid] if split_edges else dstr.at[sid]
        for ci in range(chunk_iters):
            c = 0 if split_edges else 2 * ci + cid
            o = cid if split_edges else c
            tab_c = tab.at[c]
            pltpu.sync_copy(zeros.at[pl.ds(r0, ROWS_PER_TILE)],
                            accum.at[pl.ds(r0, ROWS_PER_TILE)])
            plsc.subcore_barrier()

            for off, rb in blocks:
                pltpu.sync_copy(my_src.at[pl.ds(off, rb)],
                                src_v.at[pl.ds(0, rb)])
                pltpu.sync_copy(my_dst.at[pl.ds(off, rb)],
                                dst_v.at[pl.ds(0, rb)])
                pltpu.async_copy(tab_c.at[src_v.at[0]], rows0, sem0)

                def body(i, _, rb=rb):
                    u = 2 * i
                    pltpu.async_copy(tab_c.at[src_v.at[u + 1]], rows1, sem1)
                    pltpu.make_async_copy(tab_c.at[src_v.at[u]], rows0,
                                          sem0).wait()
                    pltpu.sync_copy(rows0, accum.at[dst_v.at[u]], add=True)

                    @pl.when(u + 2 < rb)
                    def _():
                        pltpu.async_copy(tab_c.at[src_v.at[u + 2]], rows0, sem0)

                    pltpu.make_async_copy(tab_c.at[src_v.at[u + 1]], rows1,
                                          sem1).wait()
                    pltpu.sync_copy(rows1, accum.at[dst_v.at[u + 1]], add=True)
                    return 0

                lax.fori_loop(0, rb // 2, body, 0)
            plsc.subcore_barrier()
            pltpu.sync_copy(accum.at[pl.ds(r0, ROWS_PER_TILE)],
                            out.at[o].at[pl.ds(r0, ROWS_PER_TILE)])
            plsc.subcore_barrier()

    return g_kernel


@functools.cache
def _make_deg_kernel():
    @functools.partial(
        pl.kernel,
        out_type=(jax.ShapeDtypeStruct((2, NP), F32),
                  jax.ShapeDtypeStruct((2, NP), F32)),
        mesh=plsc.VectorSubcoreMesh(**_SC_MESH),
        scratch_types=[
            pltpu.VMEM((RD, 128), jnp.int32),
            pltpu.VMEM((RD, 128), jnp.int32),
            pltpu.VMEM((RD, 128), F32),
            pltpu.VMEM((RD, 128), F32),
            pltpu.VMEM_SHARED((NP,), F32),
            pltpu.VMEM_SHARED((NP,), F32),
        ],
    )
    def deg_kernel(srcr, dstr, zeros1, deg_out, cs_out, src_v, dst_v,
                   mval, cval, acc_deg, acc_cs):
        cid = lax.axis_index("c")
        sid = lax.axis_index("s")
        wid = sid * 2 + cid
        r0 = sid * ROWS_PER_TILE
        pltpu.sync_copy(srcr.at[wid], src_v)
        pltpu.sync_copy(dstr.at[wid], dst_v)
        pltpu.sync_copy(zeros1.at[pl.ds(r0, ROWS_PER_TILE)],
                        acc_deg.at[pl.ds(r0, ROWS_PER_TILE)])
        pltpu.sync_copy(zeros1.at[pl.ds(r0, ROWS_PER_TILE)],
                        acc_cs.at[pl.ds(r0, ROWS_PER_TILE)])
        plsc.subcore_barrier()

        def body(j, _):
            for i in range(8):
                s = src_v[j, pl.ds(i * 16, 16)]
                d = dst_v[j, pl.ds(i * 16, 16)]
                m = jnp.where(s != d, F32(1.0), F32(0.0))
                mval[j, pl.ds(i * 16, 16)] = m
                cval[j, pl.ds(i * 16, 16)] = F32(1.0) - m
            pltpu.sync_copy(mval.at[j], acc_deg.at[src_v.at[j]], add=True)
            pltpu.sync_copy(cval.at[j], acc_cs.at[src_v.at[j]], add=True)
            return 0

        lax.fori_loop(0, RD, body, 0)
        plsc.subcore_barrier()
        pltpu.sync_copy(acc_deg.at[pl.ds(r0, ROWS_PER_TILE)],
                        deg_out.at[cid].at[pl.ds(r0, ROWS_PER_TILE)])
        pltpu.sync_copy(acc_cs.at[pl.ds(r0, ROWS_PER_TILE)],
                        cs_out.at[cid].at[pl.ds(r0, ROWS_PER_TILE)])

    return deg_kernel


def _prologue_body(deg2_ref, cs2_ref, x_ref, dinv_ref, cs_ref, xc_ref, xh_ref):
    deg = jnp.sum(deg2_ref[...], axis=0)
    cs = jnp.sum(cs2_ref[...], axis=0)
    dinv = jnp.where(deg > 0, lax.rsqrt(jnp.maximum(deg, F32(1.0))), F32(0.0))
    dinv_ref[...] = dinv
    cs_ref[...] = cs
    x = x_ref[...]
    xc_ref[0] = x
    xh_ref[0] = dinv * x


def _prologue(deg2, cs2, xp):
    return pl.pallas_call(
        _prologue_body,
        grid=(NB,),
        in_specs=[
            pl.BlockSpec((2, 256, 1), lambda i: (0, i, 0)),
            pl.BlockSpec((2, 256, 1), lambda i: (0, i, 0)),
            pl.BlockSpec((256, 128), lambda i: (i, 0)),
        ],
        out_specs=[
            pl.BlockSpec((256, 1), lambda i: (i, 0)),
            pl.BlockSpec((256, 1), lambda i: (i, 0)),
            pl.BlockSpec((1, 256, 128), lambda i: (0, i, 0)),
            pl.BlockSpec((1, 256, 128), lambda i: (0, i, 0)),
        ],
        out_shape=[
            jax.ShapeDtypeStruct((NP, 1), F32),
            jax.ShapeDtypeStruct((NP, 1), F32),
            jax.ShapeDtypeStruct((1, NP, 128), F32),
            jax.ShapeDtypeStruct((1, NP, 128), F32),
        ],
    )(deg2, cs2, xp)


def _cheb(g, yh, tpp, dinv, cs, alpha):
    if len(g) != len(yh):
        gg = lambda c: g[0] + g[1]
    else:
        gg = lambda c: g[c]
    ts = []
    for c in range(len(yh)):
        t = (-alpha) * dinv * (gg(c) - cs * yh[c])
        if tpp is not None:
            t = t - tpp[c]
        ts.append(t)
    return ts


def _elt_body(nc, alpha, g_ref, yh_ref, tpp_ref, dinv_ref, cs_ref,
              tx_ref, yhn_ref):
    d = dinv_ref[...]
    s = cs_ref[...]
    g = [g_ref[c] for c in range(g_ref.shape[0])]
    yh = [yh_ref[c] for c in range(nc)]
    tpp = None if tpp_ref is yh_ref else [tpp_ref[c] for c in range(nc)]
    ts = _cheb(g, yh, tpp, d, s, alpha)
    for c in range(nc):
        tx_ref[c] = ts[c]
        yhn_ref[c] = d * ts[c]


def _elt(g, yh, tpp, dinv, cs, alpha):
    nc, _, W = yh.shape
    gnc = g.shape[0]

    def body(g_ref, yh_ref, tpp_ref, dinv_ref, cs_ref, tx_ref, yhn_ref):
        _elt_body(nc, alpha, g_ref, yh_ref,
                  yh_ref if tpp is None else tpp_ref,
                  dinv_ref, cs_ref, tx_ref, yhn_ref)

    return pl.pallas_call(
        body,
        grid=(NB,),
        in_specs=[
            pl.BlockSpec((gnc, 256, W), lambda i: (0, i, 0)),
            pl.BlockSpec((nc, 256, W), lambda i: (0, i, 0)),
            pl.BlockSpec((nc, 256, W), lambda i: (0, i, 0)),
            pl.BlockSpec((256, 1), lambda i: (i, 0)),
            pl.BlockSpec((256, 1), lambda i: (i, 0)),
        ],
        out_specs=[
            pl.BlockSpec((nc, 256, W), lambda i: (0, i, 0)),
            pl.BlockSpec((nc, 256, W), lambda i: (0, i, 0)),
        ],
        out_shape=[
            jax.ShapeDtypeStruct((nc, NP, W), F32),
            jax.ShapeDtypeStruct((nc, NP, W), F32),
        ],
    )(g, yh, yh if tpp is None else tpp, dinv, cs)


def _stepf_body(nc, fout, g_ref, yh_ref, tx1_ref, xc_ref, tx2_ref, dinv_ref,
                cs_ref, w_ref, b_ref, acc_ref, sums_ref, sacc_ref):
    i = pl.program_id(0)
    d = dinv_ref[...]
    s = cs_ref[...]
    acc = jnp.broadcast_to(b_ref[...], acc_ref.shape).astype(F32)
    g = [g_ref[c] for c in range(g_ref.shape[0])]
    yh = [yh_ref[c] for c in range(nc)]
    tpp = [tx1_ref[c] for c in range(nc)]
    ts = _cheb(g, yh, tpp, d, s, 2.0)
    for c in range(nc):
        acc = acc + jnp.dot(xc_ref[c], w_ref[0, c], preferred_element_type=F32)
        acc = acc + jnp.dot(tx1_ref[c], w_ref[1, c], preferred_element_type=F32)
        acc = acc + jnp.dot(tx2_ref[c], w_ref[2, c], preferred_element_type=F32)
        acc = acc + jnp.dot(ts[c], w_ref[3, c], preferred_element_type=F32)
    acc_ref[...] = acc
    row = lax.broadcasted_iota(jnp.int32, (256, 1), 0) + i * 256
    am = jnp.where(row < N, acc, F32(0.0))

    @pl.when(i == 0)
    def _():
        sacc_ref[...] = jnp.zeros(sacc_ref.shape, F32)

    sacc_ref[0:1] += jnp.sum(am, axis=0, keepdims=True)
    sacc_ref[1:2] += jnp.sum(am * am, axis=0, keepdims=True)
    sums_ref[...] = sacc_ref[0:2]


def _stepf(g, yh, tx1, xc, tx2, dinv, cs, wc, b):
    nc, _, W = yh.shape
    gnc = g.shape[0]
    fout = wc.shape[3]
    return pl.pallas_call(
        functools.partial(_stepf_body, nc, fout),
        grid=(NB,),
        in_specs=[
            pl.BlockSpec((gnc, 256, W), lambda i: (0, i, 0)),
            pl.BlockSpec((nc, 256, W), lambda i: (0, i, 0)),
            pl.BlockSpec((nc, 256, W), lambda i: (0, i, 0)),
            pl.BlockSpec((nc, 256, W), lambda i: (0, i, 0)),
            pl.BlockSpec((nc, 256, W), lambda i: (0, i, 0)),
            pl.BlockSpec((256, 1), lambda i: (i, 0)),
            pl.BlockSpec((256, 1), lambda i: (i, 0)),
            pl.BlockSpec((4, nc, W, fout), lambda i: (0, 0, 0, 0)),
            pl.BlockSpec((1, fout), lambda i: (0, 0)),
        ],
        out_specs=[
            pl.BlockSpec((256, fout), lambda i: (i, 0)),
            pl.BlockSpec((2, fout), lambda i: (0, 0)),
        ],
        out_shape=[
            jax.ShapeDtypeStruct((NP, fout), F32),
            jax.ShapeDtypeStruct((2, fout), F32),
        ],
        scratch_shapes=[pltpu.VMEM((8, fout), F32)],
    )(g, yh, tx1, xc, tx2, dinv, cs, wc, b.reshape(1, fout))


def _gnorm_y(acc_ref, sums_ref, gw_ref, gb_ref, gm_ref):
    a = acc_ref[...]
    inv_n = F32(1.0 / N)
    mean = sums_ref[0:1] * inv_n
    ex2 = sums_ref[1:2] * inv_n
    mm = mean * gm_ref[...]
    var = ex2 - 2.0 * mm * mean + mm * mm
    std = lax.sqrt(var + F32(1e-5))
    return gw_ref[...] * (a - mm) / std + gb_ref[...]


def _norm_body(fout, ncn, acc_ref, sums_ref, gw_ref, gb_ref, gm_ref,
               dinv_ref, out0_ref, out1_ref):
    y = _gnorm_y(acc_ref, sums_ref, gw_ref, gb_ref, gm_ref)
    y = jnp.where(y >= 0, y, F32(0.1) * y)
    d = dinv_ref[...]
    Wn = fout // ncn
    for c in range(ncn):
        ys = y[:, c * Wn:(c + 1) * Wn]
        out0_ref[c] = ys
        out1_ref[c] = d * ys


def _norm(acc, sums, gw, gb, gm, dinv, ncn):
    fout = acc.shape[1]
    Wn = fout // ncn
    return pl.pallas_call(
        functools.partial(_norm_body, fout, ncn),
        grid=(NB,),
        in_specs=[
            pl.BlockSpec((256, fout), lambda i: (i, 0)),
            pl.BlockSpec((2, fout), lambda i: (0, 0)),
            pl.BlockSpec((1, fout), lambda i: (0, 0)),
            pl.BlockSpec((1, fout), lambda i: (0, 0)),
            pl.BlockSpec((1, fout), lambda i: (0, 0)),
            pl.BlockSpec((256, 1), lambda i: (i, 0)),
        ],
        out_specs=[
            pl.BlockSpec((ncn, 256, Wn), lambda i: (0, i, 0)),
            pl.BlockSpec((ncn, 256, Wn), lambda i: (0, i, 0)),
        ],
        out_shape=[
            jax.ShapeDtypeStruct((ncn, NP, Wn), F32),
            jax.ShapeDtypeStruct((ncn, NP, Wn), F32),
        ],
    )(acc, sums, gw.reshape(1, fout), gb.reshape(1, fout),
      gm.reshape(1, fout), dinv)


def _tail_body(fout, acc_ref, sums_ref, gw_ref, gb_ref, gm_ref, x_ref,
               w1_ref, b1_ref, w2_ref, b2_ref, o_ref, pacc_ref):
    i = pl.program_id(0)
    y = _gnorm_y(acc_ref, sums_ref, gw_ref, gb_ref, gm_ref)
    row = lax.broadcasted_iota(jnp.int32, (256, 1), 0) + i * 256
    h = jnp.maximum(y + x_ref[...], F32(0.0))
    hm = jnp.where(row < N, h, F32(0.0))

    @pl.when(i == 0)
    def _():
        pacc_ref[...] = jnp.zeros(pacc_ref.shape, F32)

    pacc_ref[0:1] += jnp.sum(hm, axis=0, keepdims=True)

    @pl.when(i == NB - 1)
    def _():
        pooled = pacc_ref[0:1] * F32(1.0 / N)
        hh = jnp.tanh(jnp.dot(pooled, w1_ref[...],
                              preferred_element_type=F32) + b1_ref[...])
        o_ref[...] = jnp.dot(hh, w2_ref[...],
                             preferred_element_type=F32) + b2_ref[...]

    @pl.when(i < NB - 1)
    def _():
        o_ref[...] = jnp.zeros(o_ref.shape, F32)


def _tail(acc, sums, gw, gb, gm, xp, l1W, l1b, l2W, l2b):
    fout = acc.shape[1]
    return pl.pallas_call(
        functools.partial(_tail_body, fout),
        grid=(NB,),
        in_specs=[
            pl.BlockSpec((256, fout), lambda i: (i, 0)),
            pl.BlockSpec((2, fout), lambda i: (0, 0)),
            pl.BlockSpec((1, fout), lambda i: (0, 0)),
            pl.BlockSpec((1, fout), lambda i: (0, 0)),
            pl.BlockSpec((1, fout), lambda i: (0, 0)),
            pl.BlockSpec((256, 128), lambda i: (i, 0)),
            pl.BlockSpec((128, 64), lambda i: (0, 0)),
            pl.BlockSpec((1, 64), lambda i: (0, 0)),
            pl.BlockSpec((64, 12), lambda i: (0, 0)),
            pl.BlockSpec((1, 12), lambda i: (0, 0)),
        ],
        out_specs=pl.BlockSpec((1, 12), lambda i: (0, 0)),
        out_shape=jax.ShapeDtypeStruct((1, 12), F32),
        scratch_shapes=[pltpu.VMEM((8, fout), F32)],
    )(acc, sums, gw.reshape(1, fout), gb.reshape(1, fout),
      gm.reshape(1, fout), xp, l1W, l1b.reshape(1, 64), l2W,
      l2b.reshape(1, 12))


_CHUNKS = {128: (1, 128), 256: (2, 128), 512: (4, 128)}


def kernel(x, edge_index, batch, W1, b1, g1w, g1b, g1m, W2, b2, g2w, g2b, g2m,
           W3, b3, g3w, g3b, g3m, W4, b4, g4w, g4b, g4m, l1W, l1b, l2W, l2b):
    del batch
    src = edge_index[0].astype(jnp.int32)
    dst = edge_index[1].astype(jnp.int32)
    pad_idx = (N + jnp.arange(EP - E, dtype=jnp.int32) % (NP - N))
    src_p = jnp.concatenate([src, pad_idx])
    dst_p = jnp.concatenate([dst, pad_idx])
    src16 = src_p.reshape(16, RG, 128)
    dst16 = dst_p.reshape(16, RG, 128)
    src32 = src_p.reshape(32, RD, 128)
    dst32 = dst_p.reshape(32, RD, 128)
    zeros2 = jnp.zeros((NP, 128), F32)
    zeros1 = jnp.zeros((NP,), F32)
    xp = jnp.pad(x, ((0, NP - N), (0, 0)))

    deg2, cs2 = _make_deg_kernel()(src32, dst32, zeros1)
    dinv, csl, xc, xh = _prologue(deg2.reshape(2, NP, 1), cs2.reshape(2, NP, 1), xp)

    layers = [
        (W1, b1, g1w, g1b, g1m),
        (W2, b2, g2w, g2b, g2m),
        (W3, b3, g3w, g3b, g3m),
        (W4, b4, g4w, g4b, g4m),
    ]
    pooled = None
    for li, (Wl, bl, gw, gb, gm) in enumerate(layers):
        K, fin, fout = Wl.shape
        nc, W = _CHUNKS[fin]
        wc = Wl.reshape(K, nc, W, fout)
        g_call = _make_g_kernel(nc)
        e_src = src32 if nc == 1 else src16
        e_dst = dst32 if nc == 1 else dst16
        g1 = g_call(xh, e_src, e_dst, zeros2)
        tx1, yh1 = _elt(g1, xh, None, dinv, csl, 1.0)
        g2 = g_call(yh1, e_src, e_dst, zeros2)
        tx2, yh2 = _elt(g2, yh1, xc, dinv, csl, 2.0)
        g3 = g_call(yh2, e_src, e_dst, zeros2)
        acc, sums = _stepf(g3, yh2, tx1, xc, tx2, dinv, csl, wc, bl)
        if li < 3:
            ncn, Wn = _CHUNKS[fout]
            xc, xh = _norm(acc, sums, gw, gb, gm, dinv, ncn)
        else:
            out = _tail(acc, sums, gw, gb, gm, xp, l1W, l1b, l2W, l2b)

    return out
